# baseline (speedup 1.0000x reference)
import functools

import jax
import jax.numpy as jnp
from jax import lax
from jax.experimental import pallas as pl
from jax.experimental.pallas import tpu as pltpu
from jax.experimental.pallas import tpu_sc as plsc

VOCAB = 1000000
D = 64
SEQ = 200
BATCH = 1024
NC, NS = 2, 16
NW = NC * NS
SEQ_PER_W = BATCH // NW
NBUF = 3
GROUP_STARTS = tuple(range(0, SEQ - 16, 16)) + (SEQ - 16,)

_mesh = plsc.VectorSubcoreMesh(core_axis_name="c", subcore_axis_name="s")


def _positional_encoding(seq_len, dim):
    pos = jnp.arange(seq_len, dtype=jnp.float32)[:, None]
    half_idx = jnp.arange(dim // 2, dtype=jnp.float32)
    rates = jnp.power(10000.0, -2.0 * half_idx / float(dim))
    ang = pos * rates[None, :]
    pe = jnp.stack([jnp.sin(ang), jnp.cos(ang)], axis=-1)
    return pe.reshape(seq_len, dim)


@functools.partial(
    pl.kernel,
    out_type=jax.ShapeDtypeStruct((BATCH, SEQ, D), jnp.float32),
    mesh=_mesh,
    scratch_types=[
        pltpu.VMEM((SEQ_PER_W, SEQ), jnp.int32),
        pltpu.VMEM((NBUF, SEQ, D), jnp.float32),
        pltpu.VMEM((SEQ, D), jnp.float32),
        pltpu.SemaphoreType.DMA((NBUF,)),
        pltpu.SemaphoreType.DMA((NBUF,)),
    ],
)
def _sc_embed(x_hbm, pe_hbm, table_hbm, out_hbm,
              idx_v, rows_v, pe_v, sem_g, sem_w):
    cid = lax.axis_index("c")
    sid = lax.axis_index("s")
    wid = sid * NC + cid
    seq0 = wid * SEQ_PER_W

    table_hbm = table_hbm.reshape(VOCAB // 8, 8, D)
    pltpu.sync_copy(pe_hbm, pe_v)
    pltpu.sync_copy(x_hbm.at[pl.ds(seq0, SEQ_PER_W)], idx_v)

    def issue_gathers(s, b):
        for base in GROUP_STARTS:
            vec = idx_v[s, pl.ds(base, 16)]
            hi = lax.shift_right_logical(vec, 3)
            lo = lax.bitwise_and(vec, 7)
            lanes = range(8, 16) if base == SEQ - 16 else range(16)
            for j in lanes:
                pltpu.async_copy(
                    table_hbm.at[hi[j], lo[j]], rows_v.at[b, base + j],
                    sem_g.at[b])

    issue_gathers(0, 0)

    def phase(s, carry):
        b = lax.rem(s, NBUF)
        nb = lax.rem(s + 1, NBUF)

        @pl.when(s < SEQ_PER_W - 1)
        def _():
            @pl.when(s >= NBUF - 1)
            def _():
                pltpu.make_async_copy(
                    rows_v.at[nb], out_hbm.at[0], sem_w.at[nb]).wait()
            issue_gathers(s + 1, nb)

        pltpu.make_async_copy(
            out_hbm.at[0], rows_v.at[b], sem_g.at[b]).wait()

        def vadd(q, c):
            for d in range(4):
                sl = pl.ds(d * 16, 16)
                rows_v[b, q, sl] = rows_v[b, q, sl] + pe_v[q, sl]
            return c

        lax.fori_loop(0, SEQ, vadd, 0)

        pltpu.async_copy(rows_v.at[b], out_hbm.at[seq0 + s], sem_w.at[b])
        return carry

    lax.fori_loop(0, SEQ_PER_W, phase, 0)

    def final_drain(k, c):
        b = lax.rem(SEQ_PER_W - 1 - k, NBUF)
        pltpu.make_async_copy(rows_v.at[b], out_hbm.at[0], sem_w.at[b]).wait()
        return c

    lax.fori_loop(0, min(NBUF, SEQ_PER_W), final_drain, 0)


def kernel(x, table):
    pe = _positional_encoding(SEQ, D)
    return _sc_embed(x.astype(jnp.int32), pe, table)


# speedup vs baseline: 1.0213x; 1.0213x over previous
import functools

import jax
import jax.numpy as jnp
from jax import lax
from jax.experimental import pallas as pl
from jax.experimental.pallas import tpu as pltpu
from jax.experimental.pallas import tpu_sc as plsc

VOCAB = 1000000
D = 64
SEQ = 200
BATCH = 1024
NC, NS = 2, 16
NW = NC * NS
SEQ_PER_W = BATCH // NW
NBUF = 3
GROUP_STARTS = tuple(range(0, SEQ - 16, 16)) + (SEQ - 16,)

_mesh = plsc.VectorSubcoreMesh(core_axis_name="c", subcore_axis_name="s")


def _positional_encoding(seq_len, dim):
    pos = jnp.arange(seq_len, dtype=jnp.float32)[:, None]
    half_idx = jnp.arange(dim // 2, dtype=jnp.float32)
    rates = jnp.power(10000.0, -2.0 * half_idx / float(dim))
    ang = pos * rates[None, :]
    pe = jnp.stack([jnp.sin(ang), jnp.cos(ang)], axis=-1)
    return pe.reshape(seq_len, dim)


@functools.partial(
    pl.kernel,
    out_type=jax.ShapeDtypeStruct((BATCH, SEQ, D), jnp.float32),
    mesh=_mesh,
    scratch_types=[
        pltpu.VMEM((SEQ_PER_W, SEQ), jnp.int32),
        pltpu.VMEM((NBUF, SEQ, D), jnp.float32),
        pltpu.VMEM((SEQ, D), jnp.float32),
        pltpu.SemaphoreType.DMA((NBUF,)),
        pltpu.SemaphoreType.DMA((NBUF,)),
    ],
)
def _sc_embed(x_hbm, pe_hbm, table_hbm, out_hbm,
              idx_v, rows_v, pe_v, sem_g, sem_w):
    cid = lax.axis_index("c")
    sid = lax.axis_index("s")
    wid = sid * NC + cid
    seq0 = wid * SEQ_PER_W

    pltpu.sync_copy(pe_hbm, pe_v)
    pltpu.sync_copy(x_hbm.at[pl.ds(seq0, SEQ_PER_W)], idx_v)

    def issue_gathers(s, b):
        for base in GROUP_STARTS:
            vec = idx_v[s, pl.ds(base, 16)]
            lanes = range(8, 16) if base == SEQ - 16 else range(16)
            for j in lanes:
                r = vec[j]
                pltpu.async_copy(
                    table_hbm.at[r], rows_v.at[b, base + j], sem_g.at[b])

    issue_gathers(0, 0)

    def phase(s, carry):
        b = lax.rem(s, NBUF)
        nb = lax.rem(s + 1, NBUF)

        @pl.when(s < SEQ_PER_W - 1)
        def _():
            @pl.when(s >= NBUF - 1)
            def _():
                pltpu.make_async_copy(
                    rows_v.at[nb], out_hbm.at[0], sem_w.at[nb]).wait()
            issue_gathers(s + 1, nb)

        pltpu.make_async_copy(
            out_hbm.at[0], rows_v.at[b], sem_g.at[b]).wait()

        def vadd(g, c):
            for u in range(2):
                q = 2 * g + u
                for d in range(4):
                    sl = pl.ds(d * 16, 16)
                    rows_v[b, q, sl] = rows_v[b, q, sl] + pe_v[q, sl]
            return c

        lax.fori_loop(0, SEQ // 2, vadd, 0)

        pltpu.async_copy(rows_v.at[b], out_hbm.at[seq0 + s], sem_w.at[b])
        return carry

    lax.fori_loop(0, SEQ_PER_W, phase, 0)

    def final_drain(k, c):
        b = lax.rem(SEQ_PER_W - 1 - k, NBUF)
        pltpu.make_async_copy(rows_v.at[b], out_hbm.at[0], sem_w.at[b]).wait()
        return c

    lax.fori_loop(0, min(NBUF, SEQ_PER_W), final_drain, 0)


def kernel(x, table):
    pe = _positional_encoding(SEQ, D)
    return _sc_embed(x.astype(jnp.int32), pe, table)
